# Initial kernel scaffold; baseline (speedup 1.0000x reference)
import functools

import jax
import jax.numpy as jnp
from jax import lax
from jax.experimental import pallas as pl
from jax.experimental.pallas import tpu as pltpu
from jax.experimental.pallas import tpu_sc as plsc

VOCAB = 1000000
MAXLEN = 200
EMBED = 32
BATCH = 4096

NC, NS, L = 2, 16, 16
NW = NC * NS
TOKENS = BATCH * MAXLEN
TOK_W = TOKENS // NW
CHUNK = 1600
NCHUNK = TOK_W // CHUNK
GSUB = 64
NGATHER = CHUNK // GSUB


def _body(x_hbm, tab_hbm, pos_hbm, out_hbm, idx_v, rows_v, pos_v, sem):
    wid = lax.axis_index("s") * NC + lax.axis_index("c")
    base = wid * TOK_W

    pltpu.sync_copy(pos_hbm, pos_v)

    def chunk_body(c, _):
        tok0 = base + c * CHUNK
        pltpu.sync_copy(x_hbm.at[pl.ds(tok0 // GSUB, NGATHER)], idx_v)
        handles = []
        for j in range(NGATHER):
            handles.append(
                pltpu.async_copy(
                    tab_hbm.at[idx_v.at[j]],
                    rows_v.at[pl.ds(j * GSUB, GSUB)],
                    sem,
                )
            )
        for h in handles:
            h.wait()

        def add_body(t, _):
            p = lax.rem(t, MAXLEN)
            rows_v[t, pl.ds(0, L)] = rows_v[t, pl.ds(0, L)] + pos_v[p, pl.ds(0, L)]
            rows_v[t, pl.ds(L, L)] = rows_v[t, pl.ds(L, L)] + pos_v[p, pl.ds(L, L)]
            return 0

        lax.fori_loop(0, CHUNK, add_body, 0)

        pltpu.sync_copy(rows_v, out_hbm.at[pl.ds(tok0, CHUNK)])
        return 0

    lax.fori_loop(0, NCHUNK, chunk_body, 0)


def kernel(x, token_table, pos_table):
    x2 = x.astype(jnp.int32).reshape(TOKENS // GSUB, GSUB)
    mesh = plsc.VectorSubcoreMesh(core_axis_name="c", subcore_axis_name="s")
    run = pl.kernel(
        _body,
        mesh=mesh,
        out_type=jax.ShapeDtypeStruct((TOKENS, EMBED), jnp.float32),
        scratch_types=[
            pltpu.VMEM((NGATHER, GSUB), jnp.int32),
            pltpu.VMEM((CHUNK, EMBED), jnp.float32),
            pltpu.VMEM((MAXLEN, EMBED), jnp.float32),
            pltpu.SemaphoreType.DMA,
        ],
    )
    out = run(x2, token_table, pos_table)
    return out.reshape(BATCH, MAXLEN, EMBED)


# speedup vs baseline: 1.1897x; 1.1897x over previous
import functools

import jax
import jax.numpy as jnp
from jax import lax
from jax.experimental import pallas as pl
from jax.experimental.pallas import tpu as pltpu
from jax.experimental.pallas import tpu_sc as plsc

VOCAB = 1000000
MAXLEN = 200
EMBED = 32
BATCH = 4096

NC, NS, L = 2, 16, 16
NW = NC * NS
TOKENS = BATCH * MAXLEN
TOK_W = TOKENS // NW
CHUNK = 1600
NCHUNK = TOK_W // CHUNK
GSUB = 64
NGATHER = CHUNK // GSUB


def _body(x_hbm, tab_hbm, pos_hbm, out_hbm, idx_v, rows_v, pos_v, sem):
    wid = lax.axis_index("s") * NC + lax.axis_index("c")
    base = wid * TOK_W

    pltpu.sync_copy(pos_hbm, pos_v)
    pltpu.sync_copy(
        x_hbm.at[pl.ds(pl.multiple_of(base // GSUB, 8), TOK_W // GSUB)], idx_v
    )

    def chunk_body(c, _):
        tok0 = base + c * CHUNK
        handles = []
        for j in range(NGATHER):
            handles.append(
                pltpu.async_copy(
                    tab_hbm.at[idx_v.at[c * NGATHER + j]],
                    rows_v.at[pl.ds(j * GSUB, GSUB)],
                    sem,
                )
            )
        for h in handles:
            h.wait()

        def add_body(t, _):
            p = lax.rem(t, MAXLEN)
            rows_v[t, pl.ds(0, L)] = rows_v[t, pl.ds(0, L)] + pos_v[p, pl.ds(0, L)]
            rows_v[t, pl.ds(L, L)] = rows_v[t, pl.ds(L, L)] + pos_v[p, pl.ds(L, L)]
            return 0

        lax.fori_loop(0, CHUNK, add_body, 0)

        pltpu.sync_copy(rows_v, out_hbm.at[pl.ds(pl.multiple_of(tok0, 8), CHUNK)])
        return 0

    lax.fori_loop(0, NCHUNK, chunk_body, 0)


def kernel(x, token_table, pos_table):
    x2 = x.astype(jnp.int32).reshape(TOKENS // GSUB, GSUB)
    mesh = plsc.VectorSubcoreMesh(core_axis_name="c", subcore_axis_name="s")
    run = pl.kernel(
        _body,
        mesh=mesh,
        out_type=jax.ShapeDtypeStruct((TOKENS, EMBED), jnp.float32),
        scratch_types=[
            pltpu.VMEM((TOK_W // GSUB, GSUB), jnp.int32),
            pltpu.VMEM((CHUNK, EMBED), jnp.float32),
            pltpu.VMEM((MAXLEN, EMBED), jnp.float32),
            pltpu.SemaphoreType.DMA,
        ],
        compiler_params=pltpu.CompilerParams(use_tc_tiling_on_sc=False),
    )
    out = run(x2, token_table, pos_table)
    return out.reshape(BATCH, MAXLEN, EMBED)


# speedup vs baseline: 1.2977x; 1.0908x over previous
import jax
import jax.numpy as jnp
from jax import lax
from jax.experimental import pallas as pl
from jax.experimental.pallas import tpu as pltpu
from jax.experimental.pallas import tpu_sc as plsc

VOCAB = 1000000
MAXLEN = 200
EMBED = 32
BATCH = 4096

NC, NS, L = 2, 16, 16
NW = NC * NS
TOKENS = BATCH * MAXLEN
TOK_W = TOKENS // NW
CHUNK = 800
NCHUNK = TOK_W // CHUNK
GSUB = 80
NGATHER = CHUNK // GSUB


def _body(x_hbm, tab_hbm, posrep_hbm, out_hbm,
          idx_v, rows0_v, rows1_v,
          sem_p0, sem_p1, sem_g0, sem_g1, sem_w0, sem_w1):
    wid = lax.axis_index("s") * NC + lax.axis_index("c")
    base = wid * TOK_W

    rows = (rows0_v, rows1_v)
    sem_p = (sem_p0, sem_p1)
    sem_g = (sem_g0, sem_g1)
    sem_w = (sem_w0, sem_w1)

    pltpu.sync_copy(
        x_hbm.at[pl.ds(pl.multiple_of(base // GSUB, 8), TOK_W // GSUB)], idx_v
    )

    def prefill(b):
        return pltpu.async_copy(posrep_hbm, rows[b], sem_p[b])

    def fire_gathers(b, c):
        hs = []
        for j in range(NGATHER):
            hs.append(
                pltpu.async_copy(
                    tab_hbm.at[idx_v.at[c * NGATHER + j]],
                    rows[b].at[pl.ds(j * GSUB, GSUB)],
                    sem_g[b],
                    add=True,
                )
            )
        return hs

    def writeback(b, c):
        tok0 = pl.multiple_of(base + c * CHUNK, 8)
        return pltpu.async_copy(rows[b], out_hbm.at[pl.ds(tok0, CHUNK)], sem_w[b])

    def wait_prev_writeback(b):
        pltpu.make_async_copy(rows[b], out_hbm.at[pl.ds(0, CHUNK)], sem_w[b]).wait()

    def pair_body(i, _):
        c0 = 2 * i
        c1 = 2 * i + 1

        @pl.when(i > 0)
        def _():
            wait_prev_writeback(0)

        prefill(0).wait()
        gs0 = fire_gathers(0, c0)

        @pl.when(i > 0)
        def _():
            wait_prev_writeback(1)

        prefill(1).wait()
        gs1 = fire_gathers(1, c1)

        for h in gs0:
            h.wait()
        writeback(0, c0)
        for h in gs1:
            h.wait()
        writeback(1, c1)
        return 0

    lax.fori_loop(0, NCHUNK // 2, pair_body, 0)
    wait_prev_writeback(0)
    wait_prev_writeback(1)


def kernel(x, token_table, pos_table):
    x2 = x.astype(jnp.int32).reshape(TOKENS // GSUB, GSUB)
    posrep = jnp.tile(pos_table, (CHUNK // MAXLEN, 1))
    mesh = plsc.VectorSubcoreMesh(core_axis_name="c", subcore_axis_name="s")
    run = pl.kernel(
        _body,
        mesh=mesh,
        out_type=jax.ShapeDtypeStruct((TOKENS, EMBED), jnp.float32),
        scratch_types=[
            pltpu.VMEM((TOK_W // GSUB, GSUB), jnp.int32),
            pltpu.VMEM((CHUNK, EMBED), jnp.float32),
            pltpu.VMEM((CHUNK, EMBED), jnp.float32),
            pltpu.SemaphoreType.DMA,
            pltpu.SemaphoreType.DMA,
            pltpu.SemaphoreType.DMA,
            pltpu.SemaphoreType.DMA,
            pltpu.SemaphoreType.DMA,
            pltpu.SemaphoreType.DMA,
        ],
        compiler_params=pltpu.CompilerParams(use_tc_tiling_on_sc=False),
    )
    out = run(x2, token_table, posrep)
    return out.reshape(BATCH, MAXLEN, EMBED)


# speedup vs baseline: 1.3995x; 1.0784x over previous
import jax
import jax.numpy as jnp
from jax import lax
from jax.experimental import pallas as pl
from jax.experimental.pallas import tpu as pltpu
from jax.experimental.pallas import tpu_sc as plsc

VOCAB = 1000000
MAXLEN = 200
EMBED = 32
BATCH = 4096

NC, NS, L = 2, 16, 16
NW = NC * NS
BPW = BATCH // NW
T_CH = 8
NCHUNK = MAXLEN // T_CH
EB = EMBED // 8
TILE_F = 8 * 128


def _body(xT_hbm, tab_hbm, pos_hbm, out_hbm,
          idx0_v, idx1_v, rows0_v, rows1_v, outv_v, pos_v,
          sem_g0, sem_g1, sem_w):
    wid = lax.axis_index("s") * NC + lax.axis_index("c")
    col0 = pl.multiple_of(wid * BPW, BPW)

    idxb = (idx0_v, idx1_v)
    rows = (rows0_v, rows1_v)
    sem_g = (sem_g0, sem_g1)

    pltpu.sync_copy(pos_hbm, pos_v)

    e16 = lax.iota(jnp.int32, 16)
    ip0 = ((e16 >> 3) << 10) + ((e16 & 7) << 7)
    ip1 = ip0 + 2048

    def issue_chunk(b, c):
        pltpu.sync_copy(
            xT_hbm.at[pl.ds(pl.multiple_of(c * T_CH, T_CH), T_CH),
                      pl.ds(col0, BPW)],
            idxb[b],
        )
        for k in range(T_CH):
            pltpu.async_copy(
                tab_hbm.at[idxb[b].at[k]],
                rows[b].at[pl.ds(k * BPW, BPW)],
                sem_g[b],
            )

    def drain_gathers(b):
        for _ in range(T_CH):
            pltpu.make_async_copy(
                tab_hbm.at[pl.ds(0, BPW)], rows[b].at[pl.ds(0, BPW)], sem_g[b]
            ).wait()

    def drain_out(n):
        for _ in range(n):
            pltpu.make_async_copy(
                outv_v.at[pl.ds(0, TILE_F)], out_hbm.at[0, 0, 0], sem_w
            ).wait()

    def transpose_chunk(b, c):
        for k in range(T_CH):
            t = c * T_CH + k
            pv0 = pos_v[t, pl.ds(0, L)]
            pv1 = pos_v[t, pl.ds(L, L)]

            def row_body(q, _):
                for j in range(4):
                    bl = q * 4 + j
                    r = k * BPW + bl
                    base = k * 4096 + bl
                    v0 = rows[b][r, pl.ds(0, L)] + pv0
                    v1 = rows[b][r, pl.ds(L, L)] + pv1
                    plsc.store_scatter(outv_v, [ip0 + base], v0)
                    plsc.store_scatter(outv_v, [ip1 + base], v1)
                return 0

            lax.fori_loop(0, BPW // 4, row_body, 0)

    def issue_out(c):
        for k in range(T_CH):
            t = c * T_CH + k
            for i in range(EB):
                pltpu.async_copy(
                    outv_v.at[pl.ds((k * EB + i) * 1024, 1024)],
                    out_hbm.at[t, i, wid],
                    sem_w,
                )

    issue_chunk(0, 0)
    issue_chunk(1, 1)

    def pair_body(p, _):
        c0 = 2 * p
        c1 = c0 + 1
        drain_gathers(0)

        @pl.when(p > 0)
        def _():
            drain_out(T_CH * EB)

        transpose_chunk(0, c0)
        issue_out(c0)
        issue_chunk(0, c0 + 2)
        drain_gathers(1)
        drain_out(T_CH * EB)
        transpose_chunk(1, c1)
        issue_out(c1)

        @pl.when(p < 11)
        def _():
            issue_chunk(1, c1 + 2)

        return 0

    lax.fori_loop(0, 12, pair_body, 0)
    drain_gathers(0)
    drain_out(T_CH * EB)
    transpose_chunk(0, 24)
    issue_out(24)
    drain_out(T_CH * EB)


def kernel(x, token_table, pos_table):
    xT = x.astype(jnp.int32).T
    tab_wide = lax.optimization_barrier(token_table.reshape(VOCAB // 4, EMBED * 4))
    tabL = tab_wide.reshape(VOCAB, EMBED)
    mesh = plsc.VectorSubcoreMesh(core_axis_name="c", subcore_axis_name="s")
    run = pl.kernel(
        _body,
        mesh=mesh,
        out_type=jax.ShapeDtypeStruct((MAXLEN, EB, NW, 1024), jnp.float32),
        scratch_types=[
            pltpu.VMEM((T_CH, BPW), jnp.int32),
            pltpu.VMEM((T_CH, BPW), jnp.int32),
            pltpu.VMEM((T_CH * BPW, EMBED), jnp.float32),
            pltpu.VMEM((T_CH * BPW, EMBED), jnp.float32),
            pltpu.VMEM((T_CH * EB * 1024,), jnp.float32),
            pltpu.VMEM((MAXLEN, EMBED), jnp.float32),
            pltpu.SemaphoreType.DMA,
            pltpu.SemaphoreType.DMA,
            pltpu.SemaphoreType.DMA,
        ],
        compiler_params=pltpu.CompilerParams(
            use_tc_tiling_on_sc=False, needs_layout_passes=False
        ),
    )
    out4 = run(xT, tabL, pos_table)
    out5 = out4.reshape(MAXLEN, EB, NW, 8, 128)
    return out5.transpose(2, 4, 0, 1, 3).reshape(BATCH, MAXLEN, EMBED)


# speedup vs baseline: 1.5594x; 1.1143x over previous
import jax
import jax.numpy as jnp
from jax import lax
from jax.experimental import pallas as pl
from jax.experimental.pallas import tpu as pltpu
from jax.experimental.pallas import tpu_sc as plsc

VOCAB = 1000000
MAXLEN = 200
EMBED = 32
BATCH = 4096

NC, NS, L = 2, 16, 16
NW = NC * NS
BPW = BATCH // NW
T_CH = 8
NCHUNK = MAXLEN // T_CH
EB = EMBED // 8
TILE_F = 8 * 128


def _body(xT_hbm, tab_hbm, pos_hbm, out_hbm,
          idx0_v, idx1_v, rows0_v, rows1_v, outv_v, pos_v,
          sem_g0, sem_g1, sem_w):
    wid = lax.axis_index("s") * NC + lax.axis_index("c")
    col0 = pl.multiple_of(wid * BPW, BPW)

    idxb = (idx0_v, idx1_v)
    rows = (rows0_v, rows1_v)
    sem_g = (sem_g0, sem_g1)

    pltpu.sync_copy(pos_hbm, pos_v)

    e16 = lax.iota(jnp.int32, 16)
    ip0 = ((e16 >> 3) << 10) + ((e16 & 7) << 7)
    ip1 = ip0 + 2048

    def issue_chunk(b, c):
        pltpu.sync_copy(
            xT_hbm.at[pl.ds(pl.multiple_of(c * T_CH, T_CH), T_CH),
                      pl.ds(col0, BPW)],
            idxb[b],
        )
        for k in range(T_CH):
            pltpu.async_copy(
                tab_hbm.at[idxb[b].at[k]],
                rows[b].at[pl.ds(k * BPW, BPW)],
                sem_g[b],
            )

    def drain_gathers(b):
        for _ in range(T_CH):
            pltpu.make_async_copy(
                tab_hbm.at[pl.ds(0, BPW)], rows[b].at[pl.ds(0, BPW)], sem_g[b]
            ).wait()

    def drain_out(n):
        for _ in range(n):
            pltpu.make_async_copy(
                outv_v.at[pl.ds(0, TILE_F)], out_hbm.at[0, 0, 0], sem_w
            ).wait()

    def transpose_chunk(b, c):
        for k in range(T_CH):
            t = c * T_CH + k
            pv0 = pos_v[t, pl.ds(0, L)]
            pv1 = pos_v[t, pl.ds(L, L)]
            ipk0 = ip0 + k * 4096
            ipk1 = ip1 + k * 4096

            @plsc.parallel_loop(0, BPW, unroll=8)
            def row_body(bl):
                r = k * BPW + bl
                v0 = rows[b][r, pl.ds(0, L)] + pv0
                v1 = rows[b][r, pl.ds(L, L)] + pv1
                plsc.store_scatter(outv_v, [ipk0 + bl], v0)
                plsc.store_scatter(outv_v, [ipk1 + bl], v1)

    def issue_out(c):
        for k in range(T_CH):
            t = c * T_CH + k
            for i in range(EB):
                pltpu.async_copy(
                    outv_v.at[pl.ds((k * EB + i) * 1024, 1024)],
                    out_hbm.at[t, i, wid],
                    sem_w,
                )

    issue_chunk(0, 0)
    issue_chunk(1, 1)

    def pair_body(p, _):
        c0 = 2 * p
        c1 = c0 + 1
        drain_gathers(0)

        @pl.when(p > 0)
        def _():
            drain_out(T_CH * EB)

        transpose_chunk(0, c0)
        issue_out(c0)
        issue_chunk(0, c0 + 2)
        drain_gathers(1)
        drain_out(T_CH * EB)
        transpose_chunk(1, c1)
        issue_out(c1)

        @pl.when(p < 11)
        def _():
            issue_chunk(1, c1 + 2)

        return 0

    lax.fori_loop(0, 12, pair_body, 0)
    drain_gathers(0)
    drain_out(T_CH * EB)
    transpose_chunk(0, 24)
    issue_out(24)
    drain_out(T_CH * EB)


def kernel(x, token_table, pos_table):
    xT = x.astype(jnp.int32).T
    tab_wide = lax.optimization_barrier(token_table.reshape(VOCAB // 4, EMBED * 4))
    tabL = tab_wide.reshape(VOCAB, EMBED)
    mesh = plsc.VectorSubcoreMesh(core_axis_name="c", subcore_axis_name="s")
    run = pl.kernel(
        _body,
        mesh=mesh,
        out_type=jax.ShapeDtypeStruct((MAXLEN, EB, NW, 1024), jnp.float32),
        scratch_types=[
            pltpu.VMEM((T_CH, BPW), jnp.int32),
            pltpu.VMEM((T_CH, BPW), jnp.int32),
            pltpu.VMEM((T_CH * BPW, EMBED), jnp.float32),
            pltpu.VMEM((T_CH * BPW, EMBED), jnp.float32),
            pltpu.VMEM((T_CH * EB * 1024,), jnp.float32),
            pltpu.VMEM((MAXLEN, EMBED), jnp.float32),
            pltpu.SemaphoreType.DMA,
            pltpu.SemaphoreType.DMA,
            pltpu.SemaphoreType.DMA,
        ],
        compiler_params=pltpu.CompilerParams(
            use_tc_tiling_on_sc=False, needs_layout_passes=False
        ),
    )
    out4 = run(xT, tabL, pos_table)
    out5 = out4.reshape(MAXLEN, EB, NW, 8, 128)
    return out5.transpose(2, 4, 0, 1, 3).reshape(BATCH, MAXLEN, EMBED)


# speedup vs baseline: 1.6404x; 1.0519x over previous
import jax
import jax.numpy as jnp
from jax import lax
from jax.experimental import pallas as pl
from jax.experimental.pallas import tpu as pltpu
from jax.experimental.pallas import tpu_sc as plsc

VOCAB = 1000000
MAXLEN = 200
EMBED = 32
BATCH = 4096

NC, NS, L = 2, 16, 16
NW = NC * NS
BPW = BATCH // NW
T_CH = 8
NCHUNK = MAXLEN // T_CH
EB = EMBED // 8
TILE_F = 8 * 128


def _body(xT_hbm, tab_hbm, pos_hbm, out_hbm,
          idx0_v, idx1_v, rows0_v, rows1_v, outv_v, pos_v,
          sem_g0, sem_g1, sem_w):
    wid = lax.axis_index("s") * NC + lax.axis_index("c")
    col0 = pl.multiple_of(wid * BPW, BPW)

    idxb = (idx0_v, idx1_v)
    rows = (rows0_v, rows1_v)
    sem_g = (sem_g0, sem_g1)

    pltpu.sync_copy(pos_hbm, pos_v)

    e16 = lax.iota(jnp.int32, 16)
    ip0 = ((e16 >> 3) << 10) + ((e16 & 7) << 7)
    ip1 = ip0 + 2048

    def issue_chunk(b, c):
        pltpu.sync_copy(
            xT_hbm.at[pl.ds(pl.multiple_of(c * T_CH, T_CH), T_CH),
                      pl.ds(col0, BPW)],
            idxb[b],
        )
        for k in range(T_CH):
            pltpu.async_copy(
                tab_hbm.at[idxb[b].at[k]],
                rows[b].at[pl.ds(k * BPW, BPW)],
                sem_g[b],
            )

    def drain_gathers(b):
        for _ in range(T_CH):
            pltpu.make_async_copy(
                tab_hbm.at[pl.ds(0, BPW)], rows[b].at[pl.ds(0, BPW)], sem_g[b]
            ).wait()

    def drain_out(n):
        for _ in range(n):
            pltpu.make_async_copy(
                outv_v.at[pl.ds(0, TILE_F)], out_hbm.at[0, 0, 0], sem_w
            ).wait()

    def transpose_chunk(b, c):
        for k in range(T_CH):
            t = c * T_CH + k
            pv0 = pos_v[t, pl.ds(0, L)]
            pv1 = pos_v[t, pl.ds(L, L)]
            ipk0 = ip0 + k * 4096
            ipk1 = ip1 + k * 4096

            @plsc.parallel_loop(0, BPW, unroll=8)
            def row_body(bl):
                r = k * BPW + bl
                v0 = rows[b][r, pl.ds(0, L)] + pv0
                v1 = rows[b][r, pl.ds(L, L)] + pv1
                plsc.store_scatter(outv_v, [ipk0 + bl], v0)
                plsc.store_scatter(outv_v, [ipk1 + bl], v1)

    def issue_out(c):
        for k in range(T_CH):
            t = c * T_CH + k
            for i in range(EB):
                pltpu.async_copy(
                    outv_v.at[pl.ds((k * EB + i) * 1024, 1024)],
                    out_hbm.at[t, i, wid],
                    sem_w,
                )

    issue_chunk(0, 0)
    issue_chunk(1, 1)

    def pair_body(p, _):
        c0 = 2 * p
        c1 = c0 + 1
        drain_gathers(0)

        @pl.when(p > 0)
        def _():
            drain_out(T_CH * EB)

        transpose_chunk(0, c0)
        issue_out(c0)
        issue_chunk(0, c0 + 2)
        drain_gathers(1)
        drain_out(T_CH * EB)
        transpose_chunk(1, c1)
        issue_out(c1)

        @pl.when(p < 11)
        def _():
            issue_chunk(1, c1 + 2)

        return 0

    lax.fori_loop(0, 12, pair_body, 0)
    drain_gathers(0)
    drain_out(T_CH * EB)
    transpose_chunk(0, 24)
    issue_out(24)
    drain_out(T_CH * EB)


NTCOL = VOCAB // 128
TAIL0 = NTCOL * 128


def _conv_body(tabT_hbm, tail_hbm, out_hbm,
               in0_v, in1_v, o0_v, o1_v, tail_v,
               sem_i0, sem_i1, sem_o0, sem_o1):
    wid = lax.axis_index("s") * NC + lax.axis_index("c")
    nc = jnp.where(wid < NTCOL - (NTCOL // NW) * NW, NTCOL // NW + 1, NTCOL // NW)

    in_v = (in0_v, in1_v)
    out_v = (o0_v, o1_v)
    sem_i = (sem_i0, sem_i1)
    sem_o = (sem_o0, sem_o1)

    lane32 = lax.iota(jnp.int32, 16) * 32

    def col(j):
        return wid + NW * j

    def issue_in(b, j):
        c = col(j)
        pltpu.async_copy(
            tabT_hbm.at[pl.ds(0, 32), pl.ds(pl.multiple_of(c * 128, 128), 128)],
            in_v[b],
            sem_i[b],
        )

    def drain_in(b):
        pltpu.make_async_copy(
            tabT_hbm.at[pl.ds(0, 32), pl.ds(0, 128)], in_v[b], sem_i[b]
        ).wait()

    def issue_out(b, j):
        c = col(j)
        pltpu.async_copy(
            out_v[b], out_hbm.at[pl.ds(pl.multiple_of(c * 4096, 8), 4096)], sem_o[b]
        )

    def drain_out(b):
        pltpu.make_async_copy(
            out_v[b], out_hbm.at[pl.ds(0, 4096)], sem_o[b]
        ).wait()

    def transpose(b):
        @plsc.parallel_loop(0, 32, unroll=4)
        def _(e):
            for t0 in range(0, 128, 16):
                v = in_v[b][e, pl.ds(t0, L)]
                plsc.store_scatter(out_v[b], [lane32 + (t0 * 32 + e)], v)

    issue_in(0, 0)

    def pair_body(p, _):
        j0 = 2 * p
        j1 = j0 + 1
        drain_in(0)
        issue_in(1, j1)

        @pl.when(p > 0)
        def _():
            drain_out(0)

        transpose(0)
        issue_out(0, j0)

        @pl.when(2 * p + 2 < nc)
        def _():
            issue_in(0, j0 + 2)

        drain_in(1)

        @pl.when(p > 0)
        def _():
            drain_out(1)

        transpose(1)
        issue_out(1, j1)
        return 0

    lax.fori_loop(0, nc // 2, pair_body, 0)

    @pl.when(nc % 2 == 1)
    def _():
        drain_in(0)
        drain_out(0)
        transpose(0)
        issue_out(0, nc - 1)

    @pl.when(wid == 0)
    def _():
        pltpu.sync_copy(tail_hbm, tail_v)
        pltpu.sync_copy(
            tail_v, out_hbm.at[pl.ds((VOCAB - 64) * EMBED, 64 * EMBED)]
        )

    drain_out(0)
    drain_out(1)


def kernel(x, token_table, pos_table):
    xT = x.astype(jnp.int32).T
    mesh = plsc.VectorSubcoreMesh(core_axis_name="c", subcore_axis_name="s")
    conv = pl.kernel(
        _conv_body,
        mesh=mesh,
        out_type=jax.ShapeDtypeStruct((VOCAB * EMBED,), jnp.float32),
        scratch_types=[
            pltpu.VMEM((32, 128), jnp.float32),
            pltpu.VMEM((32, 128), jnp.float32),
            pltpu.VMEM((4096,), jnp.float32),
            pltpu.VMEM((4096,), jnp.float32),
            pltpu.VMEM((64 * EMBED,), jnp.float32),
            pltpu.SemaphoreType.DMA,
            pltpu.SemaphoreType.DMA,
            pltpu.SemaphoreType.DMA,
            pltpu.SemaphoreType.DMA,
        ],
        compiler_params=pltpu.CompilerParams(
            use_tc_tiling_on_sc=True, needs_layout_passes=False
        ),
    )
    tail = token_table[TAIL0:].reshape(-1)
    tabL = conv(token_table.T, tail).reshape(VOCAB, EMBED)
    run = pl.kernel(
        _body,
        mesh=mesh,
        out_type=jax.ShapeDtypeStruct((MAXLEN, EB, NW, 1024), jnp.float32),
        scratch_types=[
            pltpu.VMEM((T_CH, BPW), jnp.int32),
            pltpu.VMEM((T_CH, BPW), jnp.int32),
            pltpu.VMEM((T_CH * BPW, EMBED), jnp.float32),
            pltpu.VMEM((T_CH * BPW, EMBED), jnp.float32),
            pltpu.VMEM((T_CH * EB * 1024,), jnp.float32),
            pltpu.VMEM((MAXLEN, EMBED), jnp.float32),
            pltpu.SemaphoreType.DMA,
            pltpu.SemaphoreType.DMA,
            pltpu.SemaphoreType.DMA,
        ],
        compiler_params=pltpu.CompilerParams(
            use_tc_tiling_on_sc=False, needs_layout_passes=False
        ),
    )
    out4 = run(xT, tabL, pos_table)
    out5 = out4.reshape(MAXLEN, EB, NW, 8, 128)
    return out5.transpose(2, 4, 0, 1, 3).reshape(BATCH, MAXLEN, EMBED)
